# baseline (speedup 1.0000x reference)
import functools

import jax
import jax.numpy as jnp
from jax import lax
from jax.experimental import pallas as pl
from jax.experimental.pallas import tpu as pltpu
from jax.experimental.pallas import tpu_sc as plsc

_NBUF = 4
_PD = 3
_GD = 2


def _build_retile_kernel(V, D):
    info = plsc.get_sparse_core_info()
    NC, NS, LN = info.num_cores, info.num_subcores, info.num_lanes
    NW = NC * NS
    assert D % LN == 0 and LN == 16
    CU = 256
    n_units = V // CU
    rem = V - n_units * CU
    main_iters = n_units // NW
    n_extra = n_units - main_iters * NW
    assert rem % 2 == 0 and (rem // 2) % 8 == 0

    mesh = plsc.VectorSubcoreMesh(core_axis_name="c", subcore_axis_name="s")

    @functools.partial(
        pl.kernel,
        out_type=jax.ShapeDtypeStruct((V // 2, 2 * D), jnp.float32),
        mesh=mesh,
        scratch_types=[
            [pltpu.VMEM((D, CU), jnp.float32) for _ in range(2)],
            [pltpu.VMEM((CU // 2, 2 * D), jnp.float32) for _ in range(2)],
            pltpu.SemaphoreType.DMA((2,)),
            pltpu.SemaphoreType.DMA((2,)),
        ],
        compiler_params=pltpu.CompilerParams(
            use_tc_tiling_on_sc=True, needs_layout_passes=False),
    )
    def retile(tT_hbm, tail_hbm, out_hbm, inb, outb, sem_i, sem_o):
        wid = lax.axis_index("s") * NC + lax.axis_index("c")

        d_iota = lax.iota(jnp.int32, 16)
        d_consts = [d_iota + cg * 16 for cg in range(D // 16)]

        if rem:
            @pl.when(wid == NW - 1)
            def _tail():
                pltpu.sync_copy(tail_hbm, outb[1].at[pl.ds(0, rem // 2)])
                pltpu.sync_copy(
                    outb[1].at[pl.ds(0, rem // 2)],
                    out_hbm.at[pl.ds(n_units * (CU // 2), rem // 2)])

        def start_in(k, b):
            u = k * NW + wid
            pltpu.async_copy(
                tT_hbm.at[:, pl.ds(u * CU, CU)], inb[b], sem_i.at[b])

        def wait_in(b):
            pltpu.make_async_copy(
                tT_hbm.at[:, pl.ds(0, CU)], inb[b], sem_i.at[b]).wait()

        def transpose(n_rows, src, dst):
            @pl.loop(0, n_rows, unroll=8)
            def _rows(j):
                for half in range(2):
                    v_idx = jnp.full((16,), 2 * j + half, jnp.int32)
                    for cg in range(D // 16):
                        val = plsc.load_gather(src, [d_consts[cg], v_idx])
                        dst[j, pl.ds(half * D + cg * 16, 16)] = val

        def start_out(k, b):
            u = k * NW + wid
            pltpu.async_copy(
                outb[b], out_hbm.at[pl.ds(u * (CU // 2), CU // 2)],
                sem_o.at[b])

        def wait_out(b):
            pltpu.make_async_copy(
                outb[b], out_hbm.at[pl.ds(0, CU // 2)], sem_o.at[b]).wait()

        start_in(0, 0)

        @pl.loop(0, main_iters)
        def _main(k):
            b = lax.rem(k, 2)
            for bb in range(2):
                @pl.when(b == bb)
                def _body():
                    wait_in(bb)

                    @pl.when(k + 1 < main_iters)
                    def _next_in():
                        start_in(k + 1, 1 - bb)

                    @pl.when(k >= 2)
                    def _drain_out():
                        wait_out(bb)

                    transpose(CU // 2, inb[bb], outb[bb])
                    start_out(k, bb)

        for k in (main_iters - 2, main_iters - 1):
            wait_out(k % 2)

        if n_extra:
            @pl.when(wid < n_extra)
            def _extra_full():
                u = main_iters * NW + wid
                pltpu.sync_copy(tT_hbm.at[:, pl.ds(u * CU, CU)], inb[0])
                transpose(CU // 2, inb[0], outb[0])
                pltpu.sync_copy(
                    outb[0], out_hbm.at[pl.ds(u * (CU // 2), CU // 2)])

    return retile


def _build_gather_kernel(B, L, V, D):
    info = plsc.get_sparse_core_info()
    NC, NS = info.num_cores, info.num_subcores
    NW = NC * NS
    assert B % (NW * _NBUF) == 0, (B, NW)
    seqs_per_w = B // NW
    rows_per_w = seqs_per_w * L
    nchunks = seqs_per_w
    DP = 128

    splits = []
    o = 0
    while o < L:
        n = min(128, L - o)
        splits.append((o, n))
        o += n

    mesh = plsc.VectorSubcoreMesh(core_axis_name="c", subcore_axis_name="s")

    @functools.partial(
        pl.kernel,
        out_type=jax.ShapeDtypeStruct((B * L, DP), jnp.float32),
        mesh=mesh,
        scratch_types=[
            pltpu.VMEM((rows_per_w,), jnp.int32),
            [pltpu.VMEM((L, D), jnp.float32) for _ in range(_NBUF)],
            pltpu.VMEM_SHARED((L, D), jnp.float32),
            pltpu.SemaphoreType.DMA((_NBUF,)),
            pltpu.SemaphoreType.DMA((_NBUF,)),
            pltpu.SemaphoreType.DMA((_NBUF,)),
        ],
        compiler_params=pltpu.CompilerParams(use_tc_tiling_on_sc=False),
    )
    def gather_k(seq_hbm, table_hbm, pos_hbm, out_hbm, idx_v, bufs,
                 pos_sh, sem_p, sem_g, sem_w):
        wid = lax.axis_index("s") * NC + lax.axis_index("c")
        base = wid * rows_per_w

        pltpu.sync_copy(seq_hbm.at[pl.ds(base, rows_per_w)], idx_v)

        @pl.when(lax.axis_index("s") == 0)
        def _fill_shared():
            pltpu.sync_copy(pos_hbm, bufs[0])
            pltpu.sync_copy(bufs[0], pos_sh)

        plsc.subcore_barrier()

        def start_prefill(b):
            pltpu.async_copy(pos_sh, bufs[b], sem_p.at[b])

        def wait_prefill(b):
            pltpu.make_async_copy(pos_sh, bufs[b], sem_p.at[b]).wait()

        def start_gathers(c, b):
            off = c * L
            for (o, n) in splits:
                pltpu.async_copy(
                    table_hbm.at[idx_v.at[pl.ds(off + o, n)]],
                    bufs[b].at[pl.ds(o, n)],
                    sem_g.at[b],
                    add=True,
                )

        def wait_gathers(b):
            for (o, n) in splits:
                pltpu.make_async_copy(
                    table_hbm.at[idx_v.at[pl.ds(o, n)]],
                    bufs[b].at[pl.ds(o, n)],
                    sem_g.at[b],
                ).wait()

        def start_write(c, b):
            pltpu.async_copy(
                bufs[b], out_hbm.at[pl.ds(base + c * L, L), pl.ds(0, D)],
                sem_w.at[b])

        def wait_write(b):
            pltpu.make_async_copy(
                bufs[b], out_hbm.at[pl.ds(base, L), pl.ds(0, D)],
                sem_w.at[b]).wait()

        for j in range(_PD):
            start_prefill(j)
        for j in range(_GD):
            wait_prefill(j)
            start_gathers(j, j)

        @pl.loop(0, nchunks // _NBUF)
        def _main(g):
            for b in range(_NBUF):
                c = g * _NBUF + b
                wait_gathers(b)
                start_write(c, b)
                bp = (b + _PD) % _NBUF

                @pl.when(c + _PD < nchunks)
                def _prefill():
                    @pl.when(c >= 1)
                    def _drain_write():
                        wait_write(bp)

                    start_prefill(bp)

                bg = (b + _GD) % _NBUF

                @pl.when(c + _GD < nchunks)
                def _gather():
                    wait_prefill(bg)
                    start_gathers(c + _GD, bg)

        for c in range(nchunks - _PD - 1, nchunks):
            wait_write(c % _NBUF)

    return gather_k


def kernel(seq, token_table, pos_table):
    B, L = seq.shape
    V, D = token_table.shape
    n_full = (V // 128) * 128
    tail = token_table[n_full:].reshape((V - n_full) // 2, 2 * D)
    packed = _build_retile_kernel(V, D)(token_table.T, tail)
    tbl = packed.reshape(V, D)
    gather_k = _build_gather_kernel(B, L, V, D)
    out = gather_k(seq.reshape(-1).astype(jnp.int32), tbl, pos_table)
    return out[:, :D].reshape(B, L, D)


# speedup vs baseline: 1.9760x; 1.9760x over previous
import functools

import jax
import jax.numpy as jnp
from jax import lax
from jax.experimental import pallas as pl
from jax.experimental.pallas import tpu as pltpu
from jax.experimental.pallas import tpu_sc as plsc

_NBUF = 4
_PD = 3
_GD = 2


def _build_gather_kernel(B, L, V, D):
    info = plsc.get_sparse_core_info()
    NC, NS = info.num_cores, info.num_subcores
    NW = NC * NS
    assert B % (NW * _NBUF) == 0, (B, NW)
    seqs_per_w = B // NW
    rows_per_w = seqs_per_w * L
    nchunks = seqs_per_w
    DP = 128

    splits = []
    o = 0
    while o < L:
        n = min(128, L - o)
        splits.append((o, n))
        o += n

    mesh = plsc.VectorSubcoreMesh(core_axis_name="c", subcore_axis_name="s")

    @functools.partial(
        pl.kernel,
        out_type=jax.ShapeDtypeStruct((B * L, DP), jnp.float32),
        mesh=mesh,
        scratch_types=[
            pltpu.VMEM((rows_per_w,), jnp.int32),
            [pltpu.VMEM((L, D), jnp.float32) for _ in range(_NBUF)],
            pltpu.VMEM_SHARED((L, D), jnp.float32),
            pltpu.SemaphoreType.DMA((_NBUF,)),
            pltpu.SemaphoreType.DMA((_NBUF,)),
            pltpu.SemaphoreType.DMA((_NBUF,)),
        ],
        compiler_params=pltpu.CompilerParams(use_tc_tiling_on_sc=False),
    )
    def gather_k(seq_hbm, table_hbm, pos_hbm, out_hbm, idx_v, bufs,
                 pos_sh, sem_p, sem_g, sem_w):
        wid = lax.axis_index("s") * NC + lax.axis_index("c")
        base = wid * rows_per_w

        pltpu.sync_copy(seq_hbm.at[pl.ds(base, rows_per_w)], idx_v)

        @pl.when(lax.axis_index("s") == 0)
        def _fill_shared():
            pltpu.sync_copy(pos_hbm, bufs[0])
            pltpu.sync_copy(bufs[0], pos_sh)

        plsc.subcore_barrier()

        def start_prefill(b):
            pltpu.async_copy(pos_sh, bufs[b], sem_p.at[b])

        def wait_prefill(b):
            pltpu.make_async_copy(pos_sh, bufs[b], sem_p.at[b]).wait()

        def start_gathers(c, b):
            off = c * L
            for (o, n) in splits:
                pltpu.async_copy(
                    table_hbm.at[idx_v.at[pl.ds(off + o, n)]],
                    bufs[b].at[pl.ds(o, n)],
                    sem_g.at[b],
                    add=True,
                )

        def wait_gathers(b):
            for (o, n) in splits:
                pltpu.make_async_copy(
                    table_hbm.at[idx_v.at[pl.ds(o, n)]],
                    bufs[b].at[pl.ds(o, n)],
                    sem_g.at[b],
                ).wait()

        def start_write(c, b):
            pltpu.async_copy(
                bufs[b], out_hbm.at[pl.ds(base + c * L, L), pl.ds(0, D)],
                sem_w.at[b])

        def wait_write(b):
            pltpu.make_async_copy(
                bufs[b], out_hbm.at[pl.ds(base, L), pl.ds(0, D)],
                sem_w.at[b]).wait()

        for j in range(_PD):
            start_prefill(j)
        for j in range(_GD):
            wait_prefill(j)
            start_gathers(j, j)

        @pl.loop(0, nchunks // _NBUF)
        def _main(g):
            for b in range(_NBUF):
                c = g * _NBUF + b
                wait_gathers(b)
                start_write(c, b)
                bp = (b + _PD) % _NBUF

                @pl.when(c + _PD < nchunks)
                def _prefill():
                    @pl.when(c >= 1)
                    def _drain_write():
                        wait_write(bp)

                    start_prefill(bp)

                bg = (b + _GD) % _NBUF

                @pl.when(c + _GD < nchunks)
                def _gather():
                    wait_prefill(bg)
                    start_gathers(c + _GD, bg)

        for c in range(nchunks - _PD - 1, nchunks):
            wait_write(c % _NBUF)

    return gather_k


def kernel(seq, token_table, pos_table):
    B, L = seq.shape
    V, D = token_table.shape
    tflat = lax.optimization_barrier(token_table.reshape(-1))
    tbl = tflat.reshape(V, D)
    gather_k = _build_gather_kernel(B, L, V, D)
    out = gather_k(seq.reshape(-1).astype(jnp.int32), tbl, pos_table)
    return out[:, :D].reshape(B, L, D)
